# baseline (speedup 1.0000x reference)
import jax
import jax.numpy as jnp
from jax.experimental import pallas as pl
from jax.experimental.pallas import tpu as pltpu

N_IN = 256
N_OUT = 10
HIDDEN = 10
N_ROWS = 100000
BLOCK_ROWS = 2000


def _fused_kernel(x_ref, w_ref, w1_ref, b1_ref, w2_ref, b2_ref, w3_ref,
                  b3_ref, out_ref, wnew_ref):
    @pl.when(pl.program_id(0) == 0)
    def _compute_w_new():
        w = w_ref[...]
        col_sum = jnp.sum(w, axis=0, keepdims=True)
        row_sum = jnp.sum(w, axis=1, keepdims=True)
        fwd = (col_sum - w) * (1.0 / (N_IN - 1))
        bwd = (row_sum - w) * (1.0 / (N_OUT - 1))
        h1 = [
            jax.nn.relu(w * w1_ref[0, k] + fwd * w1_ref[1, k]
                        + bwd * w1_ref[2, k] + b1_ref[k])
            for k in range(HIDDEN)
        ]
        upd = jnp.full(w.shape, b3_ref[0], dtype=jnp.float32)
        for j in range(HIDDEN):
            acc = jnp.full(w.shape, b2_ref[j], dtype=jnp.float32)
            for k in range(HIDDEN):
                acc = acc + h1[k] * w2_ref[k, j]
            upd = upd + jax.nn.relu(acc) * w3_ref[j, 0]
        wnew_ref[...] = w + upd

    logits = jnp.dot(x_ref[...], wnew_ref[...],
                     preferred_element_type=jnp.float32)
    m = jnp.max(logits, axis=-1, keepdims=True)
    e = jnp.exp(logits - m)
    out_ref[...] = e / jnp.sum(e, axis=-1, keepdims=True)


def kernel(X, weight, W1, b1, W2, b2, W3, b3):
    grid = (N_ROWS // BLOCK_ROWS,)
    smem = pl.BlockSpec(memory_space=pltpu.SMEM)
    return pl.pallas_call(
        _fused_kernel,
        grid=grid,
        in_specs=[
            pl.BlockSpec((BLOCK_ROWS, N_IN), lambda i: (i, 0)),
            pl.BlockSpec((N_IN, N_OUT), lambda i: (0, 0)),
            smem, smem, smem, smem, smem, smem,
        ],
        out_specs=pl.BlockSpec((BLOCK_ROWS, N_OUT), lambda i: (i, 0)),
        out_shape=jax.ShapeDtypeStruct((N_ROWS, N_OUT), jnp.float32),
        scratch_shapes=[pltpu.VMEM((N_IN, N_OUT), jnp.float32)],
        compiler_params=pltpu.CompilerParams(
            dimension_semantics=("arbitrary",)),
    )(X, weight, W1, b1, W2, b2, W3, b3)
